# baseline (speedup 1.0000x reference)
import functools

import jax
import jax.numpy as jnp
from jax import lax
from jax.experimental import pallas as pl
from jax.experimental.pallas import tpu as pltpu
from jax.experimental.pallas import tpu_sc as plsc

E = 16
D_STATE = 768
D_ACTION = 64
HIDDEN = 256
N_TOKENS = 16384
BLK = 256
NB = N_TOKENS // BLK + E
P = NB * BLK
D_ACT_PAD = 128


def _routing_metadata(policy_indices):
    pol = policy_indices.astype(jnp.int32)
    order = jnp.argsort(pol, stable=True).astype(jnp.int32)
    counts = jnp.bincount(pol, length=E)
    off = jnp.cumsum(counts) - counts
    padded = ((counts + BLK - 1) // BLK) * BLK
    pad_off = (jnp.cumsum(padded) - padded).astype(jnp.int32)
    e_r = jnp.sort(pol)
    ranks = jnp.arange(N_TOKENS, dtype=jnp.int32)
    ppos = (pad_off[e_r] + (ranks - off[e_r])).astype(jnp.int32)
    src = jnp.zeros((P,), jnp.int32).at[ppos].set(order)
    inv = jnp.zeros((N_TOKENS,), jnp.int32).at[order].set(ppos)
    block_expert = jnp.clip(
        jnp.searchsorted(pad_off, jnp.arange(NB, dtype=jnp.int32) * BLK, side="right") - 1,
        0, E - 1).astype(jnp.int32)
    return src, inv, block_expert


NC = 2
NS = 16
NW = NC * NS


def _gather_in_body(src_hbm, lat_hbm, lat_out, idx_v, lat_v, sem1):
    wid = lax.axis_index("s") * NC + lax.axis_index("c")
    rows = P // NW
    ch = 128
    base = wid * rows
    for c in range(rows // ch):
        b = base + c * ch
        pltpu.sync_copy(src_hbm.at[pl.ds(b, ch)], idx_v)
        pltpu.async_copy(lat_hbm.at[idx_v], lat_v, sem1).wait()
        pltpu.sync_copy(lat_v, lat_out.at[pl.ds(b, ch)])


def _gather_inputs(src, latents):
    ch = 128
    fn = pl.kernel(
        _gather_in_body,
        out_type=jax.ShapeDtypeStruct((P, D_STATE), jnp.float32),
        mesh=plsc.VectorSubcoreMesh(core_axis_name="c", subcore_axis_name="s"),
        scratch_types=[
            pltpu.VMEM((ch,), jnp.int32),
            pltpu.VMEM((ch, D_STATE), jnp.float32),
            pltpu.SemaphoreType.DMA,
        ],
    )
    return fn(src, latents)


def _gather_out_body(inv_hbm, outs_hbm, out_hbm, idx_v, rows_v, sem):
    wid = lax.axis_index("s") * NC + lax.axis_index("c")
    rows = N_TOKENS // NW
    ch = 128
    base = wid * rows
    for c in range(rows // ch):
        b = base + c * ch
        pltpu.sync_copy(inv_hbm.at[pl.ds(b, ch)], idx_v)
        pltpu.async_copy(outs_hbm.at[idx_v], rows_v, sem).wait()
        pltpu.sync_copy(rows_v, out_hbm.at[pl.ds(b, ch)])


def _gather_output(inv, out_s):
    ch = 128
    fn = pl.kernel(
        _gather_out_body,
        out_type=jax.ShapeDtypeStruct((N_TOKENS, D_STATE), jnp.float32),
        mesh=plsc.VectorSubcoreMesh(core_axis_name="c", subcore_axis_name="s"),
        scratch_types=[
            pltpu.VMEM((ch,), jnp.int32),
            pltpu.VMEM((ch, D_STATE), jnp.float32),
            pltpu.SemaphoreType.DMA,
        ],
    )
    return fn(inv, out_s)


def _mlp_body(be_ref, lat_ref, act_ref, w1s_ref, w1a_ref, b1_ref, w2_ref, b2_ref, out_ref):
    h = jnp.dot(lat_ref[...], w1s_ref[0], preferred_element_type=jnp.float32)
    h = h + jnp.dot(act_ref[...], w1a_ref[0], preferred_element_type=jnp.float32)
    h = jnp.maximum(h + b1_ref[0, 0], 0.0)
    out_ref[...] = jnp.dot(h, w2_ref[0], preferred_element_type=jnp.float32) + b2_ref[0, 0]


def _grouped_mlp(block_expert, lat_s, act_s, W1s, W1a, b1, W2, b2, interpret=False):
    grid_spec = pltpu.PrefetchScalarGridSpec(
        num_scalar_prefetch=1,
        grid=(NB,),
        in_specs=[
            pl.BlockSpec((BLK, D_STATE), lambda i, be: (i, 0)),
            pl.BlockSpec((BLK, D_ACT_PAD), lambda i, be: (i, 0)),
            pl.BlockSpec((1, D_STATE, HIDDEN), lambda i, be: (be[i], 0, 0)),
            pl.BlockSpec((1, D_ACT_PAD, HIDDEN), lambda i, be: (be[i], 0, 0)),
            pl.BlockSpec((1, 1, HIDDEN), lambda i, be: (be[i], 0, 0)),
            pl.BlockSpec((1, HIDDEN, D_STATE), lambda i, be: (be[i], 0, 0)),
            pl.BlockSpec((1, 1, D_STATE), lambda i, be: (be[i], 0, 0)),
        ],
        out_specs=pl.BlockSpec((BLK, D_STATE), lambda i, be: (i, 0)),
    )
    return pl.pallas_call(
        _mlp_body,
        grid_spec=grid_spec,
        out_shape=jax.ShapeDtypeStruct((P, D_STATE), jnp.float32),
        compiler_params=pltpu.CompilerParams(
            dimension_semantics=("arbitrary",),
        ),
        interpret=interpret,
    )(block_expert, lat_s, act_s, W1s, W1a, b1, W2, b2)


def kernel(latents, policy_indices, actions, W1, b1, W2, b2):
    src, inv, block_expert = _routing_metadata(policy_indices)
    lat_s = _gather_inputs(src, latents)
    act_s = jnp.pad(jnp.take(actions, src, axis=0),
                    ((0, 0), (0, D_ACT_PAD - D_ACTION)))
    W1s = W1[:, :D_STATE, :]
    W1a = jnp.pad(W1[:, D_STATE:, :], ((0, 0), (0, D_ACT_PAD - D_ACTION), (0, 0)))
    out_s = _grouped_mlp(block_expert, lat_s, act_s, W1s, W1a,
                         b1.reshape(E, 1, HIDDEN), W2, b2.reshape(E, 1, D_STATE))
    return _gather_output(inv, out_s)


# speedup vs baseline: 1.8166x; 1.8166x over previous
import functools

import jax
import jax.numpy as jnp
from jax import lax
from jax.experimental import pallas as pl
from jax.experimental.pallas import tpu as pltpu
from jax.experimental.pallas import tpu_sc as plsc

E = 16
D_STATE = 768
D_ACTION = 64
HIDDEN = 256
N_TOKENS = 16384
BLK = 256
NB = N_TOKENS // BLK + E
P = NB * BLK
D_ACT_PAD = 128
D_X = D_STATE + D_ACT_PAD


def _routing_metadata(policy_indices):
    pol = policy_indices.astype(jnp.int32)
    order = jnp.argsort(pol, stable=True).astype(jnp.int32)
    counts = jnp.bincount(pol, length=E)
    off = jnp.cumsum(counts) - counts
    padded = ((counts + BLK - 1) // BLK) * BLK
    pad_off = (jnp.cumsum(padded) - padded).astype(jnp.int32)
    e_r = jnp.sort(pol)
    ranks = jnp.arange(N_TOKENS, dtype=jnp.int32)
    ppos = (pad_off[e_r] + (ranks - off[e_r])).astype(jnp.int32)
    fill = jnp.arange(P, dtype=jnp.int32) % N_TOKENS
    src = fill.at[ppos].set(order)
    inv = jnp.zeros((N_TOKENS,), jnp.int32).at[order].set(ppos)
    block_expert = jnp.clip(
        jnp.searchsorted(pad_off, jnp.arange(NB, dtype=jnp.int32) * BLK, side="right") - 1,
        0, E - 1).astype(jnp.int32)
    return src, inv, block_expert


NC = 2
NS = 16
NW = NC * NS


CBLK = 256


def _concat_body(lat_ref, act_ref, x_ref):
    z = jnp.zeros((CBLK, D_ACT_PAD - D_ACTION), jnp.float32)
    x_ref[...] = jnp.concatenate([lat_ref[...], act_ref[...], z], axis=1)


def _concat_inputs(latents, actions):
    return pl.pallas_call(
        _concat_body,
        grid=(N_TOKENS // CBLK,),
        in_specs=[
            pl.BlockSpec((CBLK, D_STATE), lambda i: (i, 0)),
            pl.BlockSpec((CBLK, D_ACTION), lambda i: (i, 0)),
        ],
        out_specs=pl.BlockSpec((CBLK, D_X), lambda i: (i, 0)),
        out_shape=jax.ShapeDtypeStruct((N_TOKENS, D_X), jnp.float32),
    )(latents, actions)


def _gather_in_body(src_hbm, x_hbm, x_out, idx_v, x_v, sem1):
    wid = lax.axis_index("s") * NC + lax.axis_index("c")
    rows = P // NW
    ch = 128
    base = wid * rows
    for c in range(rows // ch):
        b = base + c * ch
        pltpu.sync_copy(src_hbm.at[pl.ds(b, ch)], idx_v)
        pltpu.async_copy(x_hbm.at[idx_v], x_v, sem1).wait()
        pltpu.sync_copy(x_v, x_out.at[pl.ds(b, ch)])


def _gather_inputs(src, xcat):
    ch = 128
    fn = pl.kernel(
        _gather_in_body,
        out_type=jax.ShapeDtypeStruct((P, D_X), jnp.float32),
        mesh=plsc.VectorSubcoreMesh(core_axis_name="c", subcore_axis_name="s"),
        scratch_types=[
            pltpu.VMEM((ch,), jnp.int32),
            pltpu.VMEM((ch, D_X), jnp.float32),
            pltpu.SemaphoreType.DMA,
        ],
    )
    return fn(src, xcat)


def _gather_out_body(inv_hbm, outs_hbm, out_hbm, idx_v, rows_v, sem):
    wid = lax.axis_index("s") * NC + lax.axis_index("c")
    rows = N_TOKENS // NW
    ch = 128
    base = wid * rows
    for c in range(rows // ch):
        b = base + c * ch
        pltpu.sync_copy(inv_hbm.at[pl.ds(b, ch)], idx_v)
        pltpu.async_copy(outs_hbm.at[idx_v], rows_v, sem).wait()
        pltpu.sync_copy(rows_v, out_hbm.at[pl.ds(b, ch)])


def _gather_output(inv, out_s):
    ch = 128
    fn = pl.kernel(
        _gather_out_body,
        out_type=jax.ShapeDtypeStruct((N_TOKENS, D_STATE), jnp.float32),
        mesh=plsc.VectorSubcoreMesh(core_axis_name="c", subcore_axis_name="s"),
        scratch_types=[
            pltpu.VMEM((ch,), jnp.int32),
            pltpu.VMEM((ch, D_STATE), jnp.float32),
            pltpu.SemaphoreType.DMA,
        ],
    )
    return fn(inv, out_s)


def _mlp_body(be_ref, x_ref, w1_ref, b1_ref, w2_ref, b2_ref, out_ref):
    h = jnp.dot(x_ref[...], w1_ref[0], preferred_element_type=jnp.float32)
    h = jnp.maximum(h + b1_ref[0, 0], 0.0)
    out_ref[...] = jnp.dot(h, w2_ref[0], preferred_element_type=jnp.float32) + b2_ref[0, 0]


def _grouped_mlp(block_expert, x_s, W1c, b1, W2, b2, interpret=False):
    grid_spec = pltpu.PrefetchScalarGridSpec(
        num_scalar_prefetch=1,
        grid=(NB,),
        in_specs=[
            pl.BlockSpec((BLK, D_X), lambda i, be: (i, 0)),
            pl.BlockSpec((1, D_X, HIDDEN), lambda i, be: (be[i], 0, 0)),
            pl.BlockSpec((1, 1, HIDDEN), lambda i, be: (be[i], 0, 0)),
            pl.BlockSpec((1, HIDDEN, D_STATE), lambda i, be: (be[i], 0, 0)),
            pl.BlockSpec((1, 1, D_STATE), lambda i, be: (be[i], 0, 0)),
        ],
        out_specs=pl.BlockSpec((BLK, D_STATE), lambda i, be: (i, 0)),
    )
    return pl.pallas_call(
        _mlp_body,
        grid_spec=grid_spec,
        out_shape=jax.ShapeDtypeStruct((P, D_STATE), jnp.float32),
        compiler_params=pltpu.CompilerParams(
            dimension_semantics=("arbitrary",),
        ),
        interpret=interpret,
    )(block_expert, x_s, W1c, b1, W2, b2)


def kernel(latents, policy_indices, actions, W1, b1, W2, b2):
    src, inv, block_expert = _routing_metadata(policy_indices)
    xcat = _concat_inputs(latents, actions)
    x_s = _gather_inputs(src, xcat)
    W1c = jnp.pad(W1, ((0, 0), (0, D_X - D_STATE - D_ACTION), (0, 0)))
    out_s = _grouped_mlp(block_expert, x_s, W1c,
                         b1.reshape(E, 1, HIDDEN), W2, b2.reshape(E, 1, D_STATE))
    return _gather_output(inv, out_s)


# speedup vs baseline: 2.8604x; 1.5746x over previous
import functools

import jax
import jax.numpy as jnp
from jax import lax
from jax.experimental import pallas as pl
from jax.experimental.pallas import tpu as pltpu
from jax.experimental.pallas import tpu_sc as plsc

E = 16
D_STATE = 768
D_ACTION = 64
HIDDEN = 256
N_TOKENS = 16384
BLK = 256
NB = N_TOKENS // BLK + E
P = NB * BLK
D_ACT_PAD = 128
D_X = D_STATE + D_ACT_PAD


def _routing_metadata(policy_indices):
    pol = policy_indices.astype(jnp.int32)
    order = jnp.argsort(pol).astype(jnp.int32)
    eids = jnp.arange(E, dtype=jnp.int32)
    counts = jnp.sum((pol[:, None] == eids[None, :]).astype(jnp.int32), axis=0)
    off = jnp.cumsum(counts) - counts
    padded = ((counts + BLK - 1) // BLK) * BLK
    pad_off = (jnp.cumsum(padded) - padded).astype(jnp.int32)
    delta = pad_off - off
    e_r = jnp.sort(pol)
    onehot_r = (e_r[:, None] == eids[None, :]).astype(jnp.int32)
    ranks = jnp.arange(N_TOKENS, dtype=jnp.int32)
    ppos = (ranks + jnp.sum(onehot_r * delta[None, :], axis=1)).astype(jnp.int32)
    bstart = jnp.arange(NB, dtype=jnp.int32)[:, None] * BLK
    block_expert = jnp.clip(
        jnp.sum((pad_off[None, :] <= bstart).astype(jnp.int32), axis=1) - 1,
        0, E - 1).astype(jnp.int32)
    return order, ppos, block_expert


NC = 2
NS = 16
NW = NC * NS


CBLK = 256


def _concat_body(lat_ref, act_ref, x_ref):
    z = jnp.zeros((CBLK, D_ACT_PAD - D_ACTION), jnp.float32)
    x_ref[...] = jnp.concatenate([lat_ref[...], act_ref[...], z], axis=1)


def _concat_inputs(latents, actions):
    return pl.pallas_call(
        _concat_body,
        grid=(N_TOKENS // CBLK,),
        in_specs=[
            pl.BlockSpec((CBLK, D_STATE), lambda i: (i, 0)),
            pl.BlockSpec((CBLK, D_ACTION), lambda i: (i, 0)),
        ],
        out_specs=pl.BlockSpec((CBLK, D_X), lambda i: (i, 0)),
        out_shape=jax.ShapeDtypeStruct((N_TOKENS, D_X), jnp.float32),
    )(latents, actions)


def _permute_body(gidx_hbm, sidx_hbm, tab_hbm, out_hbm, gi_v, si_v, row_v, sem1, sem2):
    wid = lax.axis_index("s") * NC + lax.axis_index("c")
    rows = N_TOKENS // NW
    ch = 128
    base = wid * rows
    for c in range(rows // ch):
        b = base + c * ch
        pltpu.sync_copy(gidx_hbm.at[pl.ds(b, ch)], gi_v)
        pltpu.sync_copy(sidx_hbm.at[pl.ds(b, ch)], si_v)
        pltpu.async_copy(tab_hbm.at[gi_v], row_v, sem1).wait()
        pltpu.async_copy(row_v, out_hbm.at[si_v], sem2).wait()


def _permute_rows(gidx, sidx, table, out_rows, width):
    ch = 128
    fn = pl.kernel(
        _permute_body,
        out_type=jax.ShapeDtypeStruct((out_rows, width), jnp.float32),
        mesh=plsc.VectorSubcoreMesh(core_axis_name="c", subcore_axis_name="s"),
        scratch_types=[
            pltpu.VMEM((ch,), jnp.int32),
            pltpu.VMEM((ch,), jnp.int32),
            pltpu.VMEM((ch, width), jnp.float32),
            pltpu.SemaphoreType.DMA,
            pltpu.SemaphoreType.DMA,
        ],
    )
    return fn(gidx, sidx, table)


def _mlp_body(be_ref, x_ref, w1_ref, b1_ref, w2_ref, b2_ref, out_ref):
    h = jnp.dot(x_ref[...], w1_ref[0], preferred_element_type=jnp.float32)
    h = jnp.maximum(h + b1_ref[0, 0], 0.0)
    out_ref[...] = jnp.dot(h, w2_ref[0], preferred_element_type=jnp.float32) + b2_ref[0, 0]


def _grouped_mlp(block_expert, x_s, W1c, b1, W2, b2, interpret=False):
    grid_spec = pltpu.PrefetchScalarGridSpec(
        num_scalar_prefetch=1,
        grid=(NB,),
        in_specs=[
            pl.BlockSpec((BLK, D_X), lambda i, be: (i, 0)),
            pl.BlockSpec((1, D_X, HIDDEN), lambda i, be: (be[i], 0, 0)),
            pl.BlockSpec((1, 1, HIDDEN), lambda i, be: (be[i], 0, 0)),
            pl.BlockSpec((1, HIDDEN, D_STATE), lambda i, be: (be[i], 0, 0)),
            pl.BlockSpec((1, 1, D_STATE), lambda i, be: (be[i], 0, 0)),
        ],
        out_specs=pl.BlockSpec((BLK, D_STATE), lambda i, be: (i, 0)),
    )
    return pl.pallas_call(
        _mlp_body,
        grid_spec=grid_spec,
        out_shape=jax.ShapeDtypeStruct((P, D_STATE), jnp.float32),
        compiler_params=pltpu.CompilerParams(
            dimension_semantics=("arbitrary",),
        ),
        interpret=interpret,
    )(block_expert, x_s, W1c, b1, W2, b2)


def kernel(latents, policy_indices, actions, W1, b1, W2, b2):
    order, ppos, block_expert = _routing_metadata(policy_indices)
    xcat = _concat_inputs(latents, actions)
    x_s = _permute_rows(order, ppos, xcat, P, D_X)
    W1c = jnp.pad(W1, ((0, 0), (0, D_X - D_STATE - D_ACTION), (0, 0)))
    out_s = _grouped_mlp(block_expert, x_s, W1c,
                         b1.reshape(E, 1, HIDDEN), W2, b2.reshape(E, 1, D_STATE))
    return _permute_rows(ppos, order, out_s, N_TOKENS, D_STATE)


# speedup vs baseline: 3.2130x; 1.1233x over previous
import functools

import jax
import jax.numpy as jnp
from jax import lax
from jax.experimental import pallas as pl
from jax.experimental.pallas import tpu as pltpu
from jax.experimental.pallas import tpu_sc as plsc

E = 16
D_STATE = 768
D_ACTION = 64
HIDDEN = 256
N_TOKENS = 16384
BLK = 256
NB = N_TOKENS // BLK + E
P = NB * BLK
D_ACT_PAD = 128
D_XP = 512


def _routing_metadata(policy_indices):
    pol = policy_indices.astype(jnp.int32)
    order = jnp.argsort(pol).astype(jnp.int32)
    eids = jnp.arange(E, dtype=jnp.int32)
    counts = jnp.sum((pol[:, None] == eids[None, :]).astype(jnp.int32), axis=0)
    off = jnp.cumsum(counts) - counts
    padded = ((counts + BLK - 1) // BLK) * BLK
    pad_off = (jnp.cumsum(padded) - padded).astype(jnp.int32)
    delta = pad_off - off
    e_r = jnp.sort(pol)
    onehot_r = (e_r[:, None] == eids[None, :]).astype(jnp.int32)
    ranks = jnp.arange(N_TOKENS, dtype=jnp.int32)
    ppos = (ranks + jnp.sum(onehot_r * delta[None, :], axis=1)).astype(jnp.int32)
    bstart = jnp.arange(NB, dtype=jnp.int32)[:, None] * BLK
    block_expert = jnp.clip(
        jnp.sum((pad_off[None, :] <= bstart).astype(jnp.int32), axis=1) - 1,
        0, E - 1).astype(jnp.int32)
    return order, ppos, block_expert


NC = 2
NS = 16
NW = NC * NS


CBLK = 256


def _round_pack(a, b):
    ua = lax.bitcast_convert_type(a, jnp.uint32)
    ub = lax.bitcast_convert_type(b, jnp.uint32)
    hi = (ua + jnp.uint32(0x8000)) & jnp.uint32(0xFFFF0000)
    lo = (ub + jnp.uint32(0x8000)) >> jnp.uint32(16)
    return lax.bitcast_convert_type(hi | lo, jnp.float32)


def _concat_body(lat_ref, act_ref, x_ref):
    lat = lat_ref[...]
    a = lat[:, :D_XP]
    z = jnp.zeros((CBLK, D_XP - (D_STATE - D_XP) - D_ACTION), jnp.float32)
    b = jnp.concatenate([lat[:, D_XP:], act_ref[...], z], axis=1)
    x_ref[...] = _round_pack(a, b)


def _concat_inputs(latents, actions):
    return pl.pallas_call(
        _concat_body,
        grid=(N_TOKENS // CBLK,),
        in_specs=[
            pl.BlockSpec((CBLK, D_STATE), lambda i: (i, 0)),
            pl.BlockSpec((CBLK, D_ACTION), lambda i: (i, 0)),
        ],
        out_specs=pl.BlockSpec((CBLK, D_XP), lambda i: (i, 0)),
        out_shape=jax.ShapeDtypeStruct((N_TOKENS, D_XP), jnp.float32),
    )(latents, actions)


def _permute_body(gidx_hbm, sidx_hbm, tab_hbm, out_hbm, gi_v, si_v, row_v, sem1, sem2):
    wid = lax.axis_index("s") * NC + lax.axis_index("c")
    rows = N_TOKENS // NW
    ch = 128
    base = wid * rows
    for c in range(rows // ch):
        b = base + c * ch
        pltpu.sync_copy(gidx_hbm.at[pl.ds(b, ch)], gi_v)
        pltpu.sync_copy(sidx_hbm.at[pl.ds(b, ch)], si_v)
        pltpu.async_copy(tab_hbm.at[gi_v], row_v, sem1).wait()
        pltpu.async_copy(row_v, out_hbm.at[si_v], sem2).wait()


def _permute_rows(gidx, sidx, table, out_rows, width, dtype):
    ch = 128
    fn = pl.kernel(
        _permute_body,
        out_type=jax.ShapeDtypeStruct((out_rows, width), dtype),
        mesh=plsc.VectorSubcoreMesh(core_axis_name="c", subcore_axis_name="s"),
        scratch_types=[
            pltpu.VMEM((ch,), jnp.int32),
            pltpu.VMEM((ch,), jnp.int32),
            pltpu.VMEM((ch, width), dtype),
            pltpu.SemaphoreType.DMA,
            pltpu.SemaphoreType.DMA,
        ],
    )
    return fn(gidx, sidx, table)


def _mlp_body(be_ref, x_ref, w1_ref, w1a_ref, b1_ref, w2_ref, b2_ref, out_ref):
    u = lax.bitcast_convert_type(x_ref[...], jnp.uint32)
    a = lax.bitcast_convert_type(u & jnp.uint32(0xFFFF0000), jnp.float32
                                 ).astype(jnp.bfloat16)
    b = lax.bitcast_convert_type(u << jnp.uint32(16), jnp.float32
                                 ).astype(jnp.bfloat16)
    h = jnp.dot(a, w1_ref[0, :D_XP, :], preferred_element_type=jnp.float32)
    h = h + jnp.dot(b[:, :D_STATE - D_XP], w1_ref[0, D_XP:, :],
                    preferred_element_type=jnp.float32)
    h = h + jnp.dot(b[:, D_STATE - D_XP:D_STATE - D_XP + D_ACT_PAD], w1a_ref[0],
                    preferred_element_type=jnp.float32)
    h = jnp.maximum(h + b1_ref[0, 0], 0.0).astype(jnp.bfloat16)
    out_ref[...] = jnp.dot(h, w2_ref[0], preferred_element_type=jnp.float32) + b2_ref[0, 0]


def _grouped_mlp(block_expert, x_s, W1, W1a, b1, W2, b2, interpret=False):
    grid_spec = pltpu.PrefetchScalarGridSpec(
        num_scalar_prefetch=1,
        grid=(NB,),
        in_specs=[
            pl.BlockSpec((BLK, D_XP), lambda i, be: (i, 0)),
            pl.BlockSpec((1, D_STATE, HIDDEN), lambda i, be: (be[i], 0, 0)),
            pl.BlockSpec((1, D_ACT_PAD, HIDDEN), lambda i, be: (be[i], 0, 0)),
            pl.BlockSpec((1, 1, HIDDEN), lambda i, be: (be[i], 0, 0)),
            pl.BlockSpec((1, HIDDEN, D_STATE), lambda i, be: (be[i], 0, 0)),
            pl.BlockSpec((1, 1, D_STATE), lambda i, be: (be[i], 0, 0)),
        ],
        out_specs=pl.BlockSpec((BLK, D_STATE), lambda i, be: (i, 0)),
    )
    return pl.pallas_call(
        _mlp_body,
        grid_spec=grid_spec,
        out_shape=jax.ShapeDtypeStruct((P, D_STATE), jnp.float32),
        compiler_params=pltpu.CompilerParams(
            dimension_semantics=("arbitrary",),
        ),
        interpret=interpret,
    )(block_expert, x_s, W1, W1a, b1, W2, b2)


def kernel(latents, policy_indices, actions, W1, b1, W2, b2):
    order, ppos, block_expert = _routing_metadata(policy_indices)
    xcat = _concat_inputs(latents, actions)
    x_s = _permute_rows(order, ppos, xcat, P, D_XP, jnp.float32)
    W1b = W1.astype(jnp.bfloat16)
    W1a = jnp.pad(W1[:, D_STATE:, :], ((0, 0), (0, D_ACT_PAD - D_ACTION), (0, 0))
                  ).astype(jnp.bfloat16)
    out_s = _grouped_mlp(block_expert, x_s, W1b, W1a,
                         b1.reshape(E, 1, HIDDEN), W2.astype(jnp.bfloat16),
                         b2.reshape(E, 1, D_STATE))
    return _permute_rows(ppos, order, out_s, N_TOKENS, D_STATE, jnp.float32)


# speedup vs baseline: 3.2252x; 1.0038x over previous
import functools

import jax
import jax.numpy as jnp
from jax import lax
from jax.experimental import pallas as pl
from jax.experimental.pallas import tpu as pltpu
from jax.experimental.pallas import tpu_sc as plsc

E = 16
D_STATE = 768
D_ACTION = 64
HIDDEN = 256
N_TOKENS = 16384
BLK = 256
NB = N_TOKENS // BLK + E
P = NB * BLK
D_ACT_PAD = 128
D_XP = 512


def _routing_metadata(policy_indices):
    pol = policy_indices.astype(jnp.int32)
    order = jnp.argsort(pol).astype(jnp.int32)
    eids = jnp.arange(E, dtype=jnp.int32)
    counts = jnp.sum((pol[:, None] == eids[None, :]).astype(jnp.int32), axis=0)
    off = jnp.cumsum(counts) - counts
    padded = ((counts + BLK - 1) // BLK) * BLK
    pad_off = (jnp.cumsum(padded) - padded).astype(jnp.int32)
    delta = pad_off - off
    e_r = jnp.sort(pol)
    onehot_r = (e_r[:, None] == eids[None, :]).astype(jnp.int32)
    ranks = jnp.arange(N_TOKENS, dtype=jnp.int32)
    ppos = (ranks + jnp.sum(onehot_r * delta[None, :], axis=1)).astype(jnp.int32)
    bstart = jnp.arange(NB, dtype=jnp.int32)[:, None] * BLK
    block_expert = jnp.clip(
        jnp.sum((pad_off[None, :] <= bstart).astype(jnp.int32), axis=1) - 1,
        0, E - 1).astype(jnp.int32)
    return order, ppos, block_expert


NC = 2
NS = 16
NW = NC * NS


CBLK = 256


def _round_pack(a, b):
    ua = lax.bitcast_convert_type(a, jnp.uint32)
    ub = lax.bitcast_convert_type(b, jnp.uint32)
    hi = (ua + jnp.uint32(0x8000)) & jnp.uint32(0xFFFF0000)
    lo = (ub + jnp.uint32(0x8000)) >> jnp.uint32(16)
    return lax.bitcast_convert_type(hi | lo, jnp.float32)


def _concat_body(lat_ref, act_ref, x_ref):
    lat = lat_ref[...]
    a = lat[:, :D_XP]
    z = jnp.zeros((CBLK, D_XP - (D_STATE - D_XP) - D_ACTION), jnp.float32)
    b = jnp.concatenate([lat[:, D_XP:], act_ref[...], z], axis=1)
    x_ref[...] = _round_pack(a, b)


def _concat_inputs(latents, actions):
    return pl.pallas_call(
        _concat_body,
        grid=(N_TOKENS // CBLK,),
        in_specs=[
            pl.BlockSpec((CBLK, D_STATE), lambda i: (i, 0)),
            pl.BlockSpec((CBLK, D_ACTION), lambda i: (i, 0)),
        ],
        out_specs=pl.BlockSpec((CBLK, D_XP), lambda i: (i, 0)),
        out_shape=jax.ShapeDtypeStruct((N_TOKENS, D_XP), jnp.float32),
    )(latents, actions)


def _permute_body(gidx_hbm, sidx_hbm, tab_hbm, out_hbm, gi_v, si_v, row_v, sem1, sem2):
    wid = lax.axis_index("s") * NC + lax.axis_index("c")
    rows = N_TOKENS // NW
    ch = 128
    base = wid * rows
    for c in range(rows // ch):
        b = base + c * ch
        pltpu.sync_copy(gidx_hbm.at[pl.ds(b, ch)], gi_v)
        pltpu.sync_copy(sidx_hbm.at[pl.ds(b, ch)], si_v)
        pltpu.async_copy(tab_hbm.at[gi_v], row_v, sem1).wait()
        pltpu.async_copy(row_v, out_hbm.at[si_v], sem2).wait()


def _permute_rows(gidx, sidx, table, out_rows, width, dtype):
    ch = 128
    fn = pl.kernel(
        _permute_body,
        out_type=jax.ShapeDtypeStruct((out_rows, width), dtype),
        mesh=plsc.VectorSubcoreMesh(core_axis_name="c", subcore_axis_name="s"),
        scratch_types=[
            pltpu.VMEM((ch,), jnp.int32),
            pltpu.VMEM((ch,), jnp.int32),
            pltpu.VMEM((ch, width), dtype),
            pltpu.SemaphoreType.DMA,
            pltpu.SemaphoreType.DMA,
        ],
    )
    return fn(gidx, sidx, table)


def _mlp_body(be_ref, x_ref, w1_ref, w1a_ref, b1_ref, w2_ref, b2_ref, out_ref):
    e = be_ref[pl.program_id(0)]
    u = lax.bitcast_convert_type(x_ref[...], jnp.uint32)
    a = lax.bitcast_convert_type(u & jnp.uint32(0xFFFF0000), jnp.float32
                                 ).astype(jnp.bfloat16)
    b = lax.bitcast_convert_type(u << jnp.uint32(16), jnp.float32
                                 ).astype(jnp.bfloat16)
    h = jnp.dot(a, w1_ref[e, :D_XP, :], preferred_element_type=jnp.float32)
    h = h + jnp.dot(b[:, :D_STATE - D_XP], w1_ref[e, D_XP:, :],
                    preferred_element_type=jnp.float32)
    h = h + jnp.dot(b[:, D_STATE - D_XP:D_STATE - D_XP + D_ACT_PAD], w1a_ref[e],
                    preferred_element_type=jnp.float32)
    h = jnp.maximum(h + b1_ref[e, 0], 0.0).astype(jnp.bfloat16)
    out_ref[...] = jnp.dot(h, w2_ref[e], preferred_element_type=jnp.float32) + b2_ref[e, 0]


def _grouped_mlp(block_expert, x_s, W1, W1a, b1, W2, b2, interpret=False):
    grid_spec = pltpu.PrefetchScalarGridSpec(
        num_scalar_prefetch=1,
        grid=(NB,),
        in_specs=[
            pl.BlockSpec((BLK, D_XP), lambda i, be: (i, 0)),
            pl.BlockSpec((E, D_STATE, HIDDEN), lambda i, be: (0, 0, 0)),
            pl.BlockSpec((E, D_ACT_PAD, HIDDEN), lambda i, be: (0, 0, 0)),
            pl.BlockSpec((E, 1, HIDDEN), lambda i, be: (0, 0, 0)),
            pl.BlockSpec((E, HIDDEN, D_STATE), lambda i, be: (0, 0, 0)),
            pl.BlockSpec((E, 1, D_STATE), lambda i, be: (0, 0, 0)),
        ],
        out_specs=pl.BlockSpec((BLK, D_STATE), lambda i, be: (i, 0)),
    )
    return pl.pallas_call(
        _mlp_body,
        grid_spec=grid_spec,
        out_shape=jax.ShapeDtypeStruct((P, D_STATE), jnp.float32),
        compiler_params=pltpu.CompilerParams(
            dimension_semantics=("arbitrary",),
        ),
        interpret=interpret,
    )(block_expert, x_s, W1, W1a, b1, W2, b2)


def kernel(latents, policy_indices, actions, W1, b1, W2, b2):
    order, ppos, block_expert = _routing_metadata(policy_indices)
    xcat = _concat_inputs(latents, actions)
    x_s = _permute_rows(order, ppos, xcat, P, D_XP, jnp.float32)
    W1b = W1[:, :D_STATE, :].astype(jnp.bfloat16)
    W1a = jnp.pad(W1[:, D_STATE:, :], ((0, 0), (0, D_ACT_PAD - D_ACTION), (0, 0))
                  ).astype(jnp.bfloat16)
    out_s = _grouped_mlp(block_expert, x_s, W1b, W1a,
                         b1.reshape(E, 1, HIDDEN), W2.astype(jnp.bfloat16),
                         b2.reshape(E, 1, D_STATE))
    return _permute_rows(ppos, order, out_s, N_TOKENS, D_STATE, jnp.float32)
